# Initial kernel scaffold; baseline (speedup 1.0000x reference)
import jax
import jax.numpy as jnp
from jax.experimental import pallas as pl


def kernel(x, edge_attr, params, edge_index, batch):
    raise NotImplementedError("write your pallas kernel here")


# speedup vs baseline: 2.5638x; 2.5638x over previous
import functools

import jax
import jax.numpy as jnp
from jax import lax
from jax.experimental import pallas as pl
from jax.experimental.pallas import tpu as pltpu
from jax.experimental.pallas import tpu_sc as plsc

_NC = 2
_NS = 16
_NW = _NC * _NS
_LANES = 16
_EBLK = 80


def _pad_cols(a, cp):
    c = a.shape[-1]
    if c == cp:
        return a
    return jnp.pad(a, [(0, 0)] * (a.ndim - 1) + [(0, cp - c)])


def _rup128(c):
    return (c + 127) // 128 * 128


def _pick_bm(m):
    for bm in (512, 400, 256, 128, 64, 32, 16, 8):
        if m % bm == 0:
            return bm
    return m


def _mm_body(act, a_ref, w_ref, b_ref, o_ref):
    acc = jnp.dot(a_ref[...], w_ref[...], preferred_element_type=jnp.float32)
    acc = acc + b_ref[...]
    if act == "elu":
        acc = jnp.where(acc > 0, acc, jnp.exp(jnp.minimum(acc, 0.0)) - 1.0)
    o_ref[...] = acc


def _mm(a, w, b=None, act=None):
    m, k = a.shape
    n = w.shape[1]
    if b is None:
        b = jnp.zeros((n,), jnp.float32)
    bm = _pick_bm(m)
    return pl.pallas_call(
        functools.partial(_mm_body, act),
        grid=(m // bm,),
        in_specs=[
            pl.BlockSpec((bm, k), lambda i: (i, 0)),
            pl.BlockSpec((k, n), lambda i: (0, 0)),
            pl.BlockSpec((1, n), lambda i: (0, 0)),
        ],
        out_specs=pl.BlockSpec((bm, n), lambda i: (i, 0)),
        out_shape=jax.ShapeDtypeStruct((m, n), jnp.float32),
    )(a, w, b.reshape(1, n))


@functools.partial(jax.jit, static_argnames=("e", "cp"))
def _sc_gather(table, idx, e, cp):
    ew = e // _NW
    nb = ew // _EBLK
    mesh = plsc.VectorSubcoreMesh(core_axis_name="c", subcore_axis_name="s")

    @functools.partial(
        pl.kernel,
        out_type=jax.ShapeDtypeStruct((e, cp), jnp.float32),
        mesh=mesh,
        scratch_types=[
            pltpu.VMEM((_EBLK,), jnp.int32),
            pltpu.VMEM((_EBLK, cp), jnp.float32),
            pltpu.SemaphoreType.DMA,
        ],
    )
    def k(table_hbm, idx_hbm, out_hbm, idx_v, rows_v, sem):
        cid = lax.axis_index("c")
        sid = lax.axis_index("s")
        base = (cid * _NS + sid) * ew

        def blk(i, carry):
            off = pl.multiple_of(base + i * _EBLK, 8)
            pltpu.sync_copy(idx_hbm.at[pl.ds(off, _EBLK)], idx_v)
            pltpu.async_copy(table_hbm.at[idx_v], rows_v, sem).wait()
            pltpu.sync_copy(rows_v, out_hbm.at[pl.ds(off, _EBLK)])
            return carry

        lax.fori_loop(0, nb, blk, 0)

    return k(table, idx)


@functools.partial(jax.jit, static_argnames=("nseg", "cp"))
def _sc_segsum(vals, idx, nseg, cp):
    e = vals.shape[0]
    ew = e // _NW
    nb = ew // _EBLK
    ch = 200
    nch = nseg // ch
    rounds = (nch + _NS - 1) // _NS
    mesh = plsc.VectorSubcoreMesh(core_axis_name="c", subcore_axis_name="s")

    @functools.partial(
        pl.kernel,
        out_type=jax.ShapeDtypeStruct((_NC, nseg, cp), jnp.float32),
        mesh=mesh,
        scratch_types=[
            pltpu.VMEM_SHARED((nseg, cp), jnp.float32),
            pltpu.VMEM((_EBLK,), jnp.int32),
            pltpu.VMEM((_EBLK, cp), jnp.float32),
            pltpu.VMEM((ch, cp), jnp.float32),
            pltpu.SemaphoreType.DMA,
        ],
    )
    def k(vals_hbm, idx_hbm, out_hbm, acc_sh, idx_v, vals_v, bounce_v, sem):
        cid = lax.axis_index("c")
        sid = lax.axis_index("s")

        zero16 = jnp.zeros((_LANES,), jnp.float32)

        def zrow(r, carry):
            for c in range(cp // _LANES):
                bounce_v[r, pl.ds(c * _LANES, _LANES)] = zero16
            return carry

        lax.fori_loop(0, ch, zrow, 0)

        def zchunk(t, carry):
            j = t * _NS + sid

            @pl.when(j < nch)
            def _():
                r0 = pl.multiple_of(j * ch, 8)
                pltpu.sync_copy(bounce_v, acc_sh.at[pl.ds(r0, ch)])

            return carry

        lax.fori_loop(0, rounds, zchunk, 0)
        plsc.subcore_barrier()

        base = (cid * _NS + sid) * ew

        def blk(i, carry):
            off = pl.multiple_of(base + i * _EBLK, 8)
            pltpu.sync_copy(idx_hbm.at[pl.ds(off, _EBLK)], idx_v)
            pltpu.sync_copy(vals_hbm.at[pl.ds(off, _EBLK)], vals_v)
            pltpu.sync_copy(vals_v, acc_sh.at[idx_v], add=True)
            return carry

        lax.fori_loop(0, nb, blk, 0)
        plsc.subcore_barrier()

        def ochunk(t, carry):
            j = t * _NS + sid

            @pl.when(j < nch)
            def _():
                r0 = pl.multiple_of(j * ch, 8)
                pltpu.sync_copy(acc_sh.at[pl.ds(r0, ch)], bounce_v)
                pltpu.sync_copy(bounce_v, out_hbm.at[cid, pl.ds(r0, ch)])

            return carry

        lax.fori_loop(0, rounds, ochunk, 0)

    parts = k(vals, idx)
    return parts[0] + parts[1]


def _segsum_wide(vals, idx, nseg):
    cp = vals.shape[1]
    parts = [_sc_segsum(vals[:, c:c + 128], idx, nseg, 128)
             for c in range(0, cp, 128)]
    return parts[0] if len(parts) == 1 else jnp.concatenate(parts, axis=1)


def kernel(x, edge_attr, params, edge_index, batch):
    n, _ = x.shape
    e = edge_index.shape[1]
    src, dst = edge_index[0], edge_index[1]
    mask = (edge_attr[:, 0:1] < 8).astype(jnp.float32)

    out = x
    n_layers = sum(1 for k_ in params if k_.startswith("conv"))
    for i in range(n_layers):
        p = params["conv%d" % i]
        cin = p["Wu"].shape[0]
        cp = _rup128(cin)
        edge = _mm(edge_attr, _pad_cols(p["We"], cp), _pad_cols(p["be"], cp),
                   act="elu")
        gx = _sc_gather(_pad_cols(out, cp), dst, e, cp)
        m = edge * gx * mask
        aggr = _segsum_wide(m, src, n)[:, :cin]
        aggr = aggr + jax.nn.elu(p["be"])[None, :] * out
        out = _mm(out + aggr, p["Wu"], p["bu"])
        g, b = params["bn%d_g" % i], params["bn%d_b" % i]
        mu = out.mean(axis=0)
        var = out.var(axis=0)
        out = g * (out - mu) / jnp.sqrt(var + 1e-5) + b

    x0 = out
    c = x0.shape[1]
    cp = _rup128(c)
    nheads = sum(1 for k_ in params if k_.startswith("att"))

    wcols, bcols = [], []
    for j in range(nheads):
        p = params["att%d" % j]
        wf1, wf2, wf3 = p["Wf"][:c], p["Wf"][c:2 * c], p["Wf"][2 * c:3 * c]
        u = p["Wk"] * p["Wa"][:, 0][None, :]
        wv = jnp.dot(p["Wq"], u.T)
        wcols += [_pad_cols(wf1 + wf3, cp), _pad_cols(wv, cp),
                  _pad_cols(wf2 - wf3, cp)]
        bcols += [jnp.zeros((3 * cp,), jnp.float32)]
    wcat = jnp.concatenate(wcols, axis=1)
    nodetab = _mm(x0, wcat, jnp.concatenate(bcols))

    w4 = jnp.concatenate(
        [_pad_cols(params["att%d" % j]["Wf"][3 * c:], cp) for j in range(nheads)],
        axis=1)
    b4 = jnp.concatenate(
        [_pad_cols(params["att%d" % j]["bf"], cp) for j in range(nheads)])
    eaf = _mm(edge_attr, w4, b4)

    scale = c ** -0.5
    f_list, a_cols, fs_list, as_cols = [], [], [], []
    for j in range(nheads):
        av = nodetab[:, j * 3 * cp:(j * 3 + 2) * cp]
        bt = nodetab[:, (j * 3 + 2) * cp:(j + 1) * 3 * cp]
        g_av = _sc_gather(av, src, e, 2 * cp)
        g_b = _sc_gather(bt, dst, e, cp)
        g_a, g_v = g_av[:, :cp], g_av[:, cp:]
        pre = g_a + g_b + eaf[:, j * cp:(j + 1) * cp]
        f = jnp.where(pre > 0, pre, jnp.expm1(pre)) * mask
        a = jnp.tanh(scale * jnp.sum(f * g_v, axis=1))
        pre_s = nodetab[:, j * 3 * cp:j * 3 * cp + cp] + bt \
            + _pad_cols(params["att%d" % j]["bf"], cp)[None, :]
        f_self = jnp.where(pre_s > 0, pre_s, jnp.expm1(pre_s))
        vtab = nodetab[:, (j * 3 + 1) * cp:(j * 3 + 2) * cp]
        a_self = jnp.tanh(scale * jnp.sum(f_self * vtab, axis=1))
        f_list.append(f)
        fs_list.append(f_self)
        a_cols.append(a)
        as_cols.append(a_self)

    a128 = jnp.zeros((e, 128), jnp.float32)
    for j in range(nheads):
        a128 = a128.at[:, j].set(a_cols[j])
    suma = _sc_segsum(a128, src, n, 128)
    for j in range(nheads):
        suma = suma.at[:, j].add(as_cols[j])
    g_suma = _sc_gather(suma, src, e, 128)

    heads = []
    for j in range(nheads):
        p = params["att%d" % j]
        z = jnp.exp(a_cols[j] - g_suma[:, j])[:, None] * f_list[j]
        aggr = _segsum_wide(z, src, n)[:, :c]
        aggr = aggr + jnp.exp(as_cols[j] - suma[:, j])[:, None] \
            * fs_list[j][:, :c]
        o = _mm(x0 + aggr, p["Wu"], p["bu"])
        g, b = params["bn2_%d_g" % j], params["bn2_%d_b" % j]
        mu = o.mean(axis=0)
        var = o.var(axis=0)
        heads.append(g * (o - mu) / jnp.sqrt(var + 1e-5) + b)

    out = jnp.concatenate(heads, axis=1)
    ngraphs = 64
    sums = jax.ops.segment_sum(out, batch, num_segments=ngraphs)
    cnt = jax.ops.segment_sum(jnp.ones((n, 1), out.dtype), batch,
                              num_segments=ngraphs)
    pooled = sums / jnp.maximum(cnt, 1.0)
    h = _mm(pooled, params["W1"], params["b1"])
    h = jnp.where(h >= 0, h, params["prelu_a"] * h)
    h = jnp.dot(h, params["W2"]) + params["b2"]
    return h.reshape(-1)


# speedup vs baseline: 3.0938x; 1.2067x over previous
import functools

import jax
import jax.numpy as jnp
from jax import lax
from jax.experimental import pallas as pl
from jax.experimental.pallas import tpu as pltpu
from jax.experimental.pallas import tpu_sc as plsc

_NC = 2
_NS = 16
_NW = _NC * _NS
_LANES = 16
_EBLK = 80


def _pad_cols(a, cp):
    c = a.shape[-1]
    if c == cp:
        return a
    return jnp.pad(a, [(0, 0)] * (a.ndim - 1) + [(0, cp - c)])


def _rup128(c):
    return (c + 127) // 128 * 128


def _pick_bm(m):
    for bm in (512, 400, 256, 128, 64, 32, 16, 8):
        if m % bm == 0:
            return bm
    return m


def _mm_body(act, a_ref, w_ref, b_ref, o_ref):
    acc = jnp.dot(a_ref[...], w_ref[...], preferred_element_type=jnp.float32)
    acc = acc + b_ref[...]
    if act == "elu":
        acc = jnp.where(acc > 0, acc, jnp.exp(jnp.minimum(acc, 0.0)) - 1.0)
    o_ref[...] = acc


def _mm(a, w, b=None, act=None):
    m, k = a.shape
    n = w.shape[1]
    if b is None:
        b = jnp.zeros((n,), jnp.float32)
    bm = _pick_bm(m)
    return pl.pallas_call(
        functools.partial(_mm_body, act),
        grid=(m // bm,),
        in_specs=[
            pl.BlockSpec((bm, k), lambda i: (i, 0)),
            pl.BlockSpec((k, n), lambda i: (0, 0)),
            pl.BlockSpec((1, n), lambda i: (0, 0)),
        ],
        out_specs=pl.BlockSpec((bm, n), lambda i: (i, 0)),
        out_shape=jax.ShapeDtypeStruct((m, n), jnp.float32),
    )(a, w, b.reshape(1, n))


@functools.partial(jax.jit, static_argnames=("e", "cp"))
def _sc_gather(table, idx, e, cp):
    ew = e // _NW
    nb = ew // _EBLK
    nbuf = 2 if cp > 256 else 4
    rounds = (nb + nbuf - 1) // nbuf
    mesh = plsc.VectorSubcoreMesh(core_axis_name="c", subcore_axis_name="s")

    @functools.partial(
        pl.kernel,
        out_type=jax.ShapeDtypeStruct((e, cp), jnp.float32),
        mesh=mesh,
        scratch_types=(
            [pltpu.VMEM((_EBLK,), jnp.int32)] * nbuf
            + [pltpu.VMEM((_EBLK, cp), jnp.float32)] * nbuf
            + [pltpu.SemaphoreType.DMA] * (3 * nbuf)
        ),
    )
    def k(table_hbm, idx_hbm, out_hbm, *scr):
        idx_v = scr[0:nbuf]
        rows_v = scr[nbuf:2 * nbuf]
        isem = scr[2 * nbuf:3 * nbuf]
        gsem = scr[3 * nbuf:4 * nbuf]
        ssem = scr[4 * nbuf:5 * nbuf]
        cid = lax.axis_index("c")
        sid = lax.axis_index("s")
        base = (cid * _NS + sid) * ew

        def ebs(i):
            return pl.ds(pl.multiple_of(base + i * _EBLK, 8), _EBLK)

        for b in range(nbuf):
            pltpu.async_copy(idx_hbm.at[ebs(b)], idx_v[b], isem[b]).wait()
            pltpu.async_copy(table_hbm.at[idx_v[b]], rows_v[b], gsem[b])

        def step(t, carry):
            for b in range(nbuf):
                i = t * nbuf + b

                @pl.when(i < nb)
                def _():
                    pltpu.make_async_copy(
                        table_hbm.at[idx_v[b]], rows_v[b], gsem[b]).wait()
                    pltpu.async_copy(rows_v[b], out_hbm.at[ebs(i)], ssem[b])
                    j = i + nbuf

                    @pl.when(j < nb)
                    def _():
                        pltpu.async_copy(
                            idx_hbm.at[ebs(j)], idx_v[b], isem[b]).wait()
                        pltpu.make_async_copy(
                            rows_v[b], out_hbm.at[ebs(i)], ssem[b]).wait()
                        pltpu.async_copy(
                            table_hbm.at[idx_v[b]], rows_v[b], gsem[b])

            return carry

        lax.fori_loop(0, rounds, step, 0)
        for b in range(nbuf):
            pltpu.make_async_copy(rows_v[b], out_hbm.at[ebs(0)], ssem[b]).wait()

    return k(table, idx)


@functools.partial(jax.jit, static_argnames=("nseg", "cp"))
def _sc_segsum(vals, idx, nseg, cp):
    e = vals.shape[0]
    ew = e // _NW
    nb = ew // _EBLK
    ch = 40
    nch = nseg // ch
    rounds = (nch + _NS - 1) // _NS
    mesh = plsc.VectorSubcoreMesh(core_axis_name="c", subcore_axis_name="s")

    nbuf = 4
    @functools.partial(
        pl.kernel,
        out_type=jax.ShapeDtypeStruct((_NC, nseg, cp), jnp.float32),
        mesh=mesh,
        scratch_types=(
            [pltpu.VMEM_SHARED((nseg, cp), jnp.float32)]
            + [pltpu.VMEM((_EBLK,), jnp.int32)] * nbuf
            + [pltpu.VMEM((_EBLK, cp), jnp.float32)] * nbuf
            + [pltpu.VMEM((ch, cp), jnp.float32)]
            + [pltpu.SemaphoreType.DMA] * (3 * nbuf)
        ),
    )
    def k(vals_hbm, idx_hbm, out_hbm, acc_sh, *scr):
        idx_v = scr[0:nbuf]
        vals_v = scr[nbuf:2 * nbuf]
        bounce_v = scr[2 * nbuf]
        isem = scr[2 * nbuf + 1:3 * nbuf + 1]
        vsem = scr[3 * nbuf + 1:4 * nbuf + 1]
        asem = scr[4 * nbuf + 1:5 * nbuf + 1]
        cid = lax.axis_index("c")
        sid = lax.axis_index("s")
        base = (cid * _NS + sid) * ew

        def ebs(i):
            return pl.ds(pl.multiple_of(base + i * _EBLK, 8), _EBLK)

        for b in range(nbuf):
            pltpu.async_copy(idx_hbm.at[ebs(b)], idx_v[b], isem[b])
            pltpu.async_copy(vals_hbm.at[ebs(b)], vals_v[b], vsem[b])

        zero16 = jnp.zeros((_LANES,), jnp.float32)

        def zrow(r, carry):
            for c in range(cp // _LANES):
                bounce_v[r, pl.ds(c * _LANES, _LANES)] = zero16
            return carry

        lax.fori_loop(0, ch, zrow, 0)

        def zchunk(t, carry):
            j = t * _NS + sid

            @pl.when(j < nch)
            def _():
                r0 = pl.multiple_of(j * ch, 8)
                pltpu.sync_copy(bounce_v, acc_sh.at[pl.ds(r0, ch)])

            return carry

        lax.fori_loop(0, rounds, zchunk, 0)
        plsc.subcore_barrier()

        def step(t, carry):
            for b in range(nbuf):
                i = t * nbuf + b

                @pl.when(i < nb)
                def _():
                    pltpu.make_async_copy(
                        idx_hbm.at[ebs(i)], idx_v[b], isem[b]).wait()
                    pltpu.make_async_copy(
                        vals_hbm.at[ebs(i)], vals_v[b], vsem[b]).wait()
                    pltpu.async_copy(
                        vals_v[b], acc_sh.at[idx_v[b]], asem[b], add=True)
                    j = i + nbuf

                    @pl.when(j < nb)
                    def _():
                        pltpu.make_async_copy(
                            vals_v[b], acc_sh.at[idx_v[b]], asem[b]).wait()
                        pltpu.async_copy(idx_hbm.at[ebs(j)], idx_v[b], isem[b])
                        pltpu.async_copy(vals_hbm.at[ebs(j)], vals_v[b], vsem[b])

            return carry

        lax.fori_loop(0, (nb + nbuf - 1) // nbuf, step, 0)
        for b in range(nbuf):
            pltpu.make_async_copy(
                vals_v[b], acc_sh.at[idx_v[b]], asem[b]).wait()
        plsc.subcore_barrier()

        def ochunk(t, carry):
            j = t * _NS + sid

            @pl.when(j < nch)
            def _():
                r0 = pl.multiple_of(j * ch, 8)
                pltpu.sync_copy(acc_sh.at[pl.ds(r0, ch)], bounce_v)
                pltpu.sync_copy(bounce_v, out_hbm.at[cid, pl.ds(r0, ch)])

            return carry

        lax.fori_loop(0, rounds, ochunk, 0)

    parts = k(vals, idx)
    return parts[0] + parts[1]


def _segsum_wide(vals, idx, nseg):
    cp = vals.shape[1]
    parts = [_sc_segsum(vals[:, c:c + 128], idx, nseg, 128)
             for c in range(0, cp, 128)]
    return parts[0] if len(parts) == 1 else jnp.concatenate(parts, axis=1)


def kernel(x, edge_attr, params, edge_index, batch):
    n, _ = x.shape
    e = edge_index.shape[1]
    src, dst = edge_index[0], edge_index[1]
    mask = (edge_attr[:, 0:1] < 8).astype(jnp.float32)

    out = x
    n_layers = sum(1 for k_ in params if k_.startswith("conv"))
    for i in range(n_layers):
        p = params["conv%d" % i]
        cin = p["Wu"].shape[0]
        cp = _rup128(cin)
        edge = _mm(edge_attr, _pad_cols(p["We"], cp), _pad_cols(p["be"], cp),
                   act="elu")
        gx = _sc_gather(_pad_cols(out, cp), dst, e, cp)
        m = edge * gx * mask
        aggr = _segsum_wide(m, src, n)[:, :cin]
        aggr = aggr + jax.nn.elu(p["be"])[None, :] * out
        out = _mm(out + aggr, p["Wu"], p["bu"])
        g, b = params["bn%d_g" % i], params["bn%d_b" % i]
        mu = out.mean(axis=0)
        var = out.var(axis=0)
        out = g * (out - mu) / jnp.sqrt(var + 1e-5) + b

    x0 = out
    c = x0.shape[1]
    cp = _rup128(c)
    nheads = sum(1 for k_ in params if k_.startswith("att"))

    wcols, bcols = [], []
    for j in range(nheads):
        p = params["att%d" % j]
        wf1, wf2, wf3 = p["Wf"][:c], p["Wf"][c:2 * c], p["Wf"][2 * c:3 * c]
        u = p["Wk"] * p["Wa"][:, 0][None, :]
        wv = jnp.dot(p["Wq"], u.T)
        wcols += [_pad_cols(wf1 + wf3, cp), _pad_cols(wv, cp),
                  _pad_cols(wf2 - wf3, cp)]
        bcols += [jnp.zeros((3 * cp,), jnp.float32)]
    wcat = jnp.concatenate(wcols, axis=1)
    nodetab = _mm(x0, wcat, jnp.concatenate(bcols))

    w4 = jnp.concatenate(
        [_pad_cols(params["att%d" % j]["Wf"][3 * c:], cp) for j in range(nheads)],
        axis=1)
    b4 = jnp.concatenate(
        [_pad_cols(params["att%d" % j]["bf"], cp) for j in range(nheads)])
    eaf = _mm(edge_attr, w4, b4)

    scale = c ** -0.5
    f_list, a_cols, fs_list, as_cols = [], [], [], []
    for j in range(nheads):
        av = nodetab[:, j * 3 * cp:(j * 3 + 2) * cp]
        bt = nodetab[:, (j * 3 + 2) * cp:(j + 1) * 3 * cp]
        g_av = _sc_gather(av, src, e, 2 * cp)
        g_b = _sc_gather(bt, dst, e, cp)
        g_a, g_v = g_av[:, :cp], g_av[:, cp:]
        pre = g_a + g_b + eaf[:, j * cp:(j + 1) * cp]
        f = jnp.where(pre > 0, pre, jnp.expm1(pre)) * mask
        a = jnp.tanh(scale * jnp.sum(f * g_v, axis=1))
        pre_s = nodetab[:, j * 3 * cp:j * 3 * cp + cp] + bt \
            + _pad_cols(params["att%d" % j]["bf"], cp)[None, :]
        f_self = jnp.where(pre_s > 0, pre_s, jnp.expm1(pre_s))
        vtab = nodetab[:, (j * 3 + 1) * cp:(j * 3 + 2) * cp]
        a_self = jnp.tanh(scale * jnp.sum(f_self * vtab, axis=1))
        f_list.append(f)
        fs_list.append(f_self)
        a_cols.append(a)
        as_cols.append(a_self)

    a128 = jnp.zeros((e, 128), jnp.float32)
    for j in range(nheads):
        a128 = a128.at[:, j].set(a_cols[j])
    suma = _sc_segsum(a128, src, n, 128)
    for j in range(nheads):
        suma = suma.at[:, j].add(as_cols[j])
    g_suma = _sc_gather(suma, src, e, 128)

    heads = []
    for j in range(nheads):
        p = params["att%d" % j]
        z = jnp.exp(a_cols[j] - g_suma[:, j])[:, None] * f_list[j]
        aggr = _segsum_wide(z, src, n)[:, :c]
        aggr = aggr + jnp.exp(as_cols[j] - suma[:, j])[:, None] \
            * fs_list[j][:, :c]
        o = _mm(x0 + aggr, p["Wu"], p["bu"])
        g, b = params["bn2_%d_g" % j], params["bn2_%d_b" % j]
        mu = o.mean(axis=0)
        var = o.var(axis=0)
        heads.append(g * (o - mu) / jnp.sqrt(var + 1e-5) + b)

    out = jnp.concatenate(heads, axis=1)
    ngraphs = 64
    sums = jax.ops.segment_sum(out, batch, num_segments=ngraphs)
    cnt = jax.ops.segment_sum(jnp.ones((n, 1), out.dtype), batch,
                              num_segments=ngraphs)
    pooled = sums / jnp.maximum(cnt, 1.0)
    h = _mm(pooled, params["W1"], params["b1"])
    h = jnp.where(h >= 0, h, params["prelu_a"] * h)
    h = jnp.dot(h, params["W2"]) + params["b2"]
    return h.reshape(-1)


# speedup vs baseline: 3.2524x; 1.0513x over previous
import functools

import jax
import jax.numpy as jnp
from jax import lax
from jax.experimental import pallas as pl
from jax.experimental.pallas import tpu as pltpu
from jax.experimental.pallas import tpu_sc as plsc

_NC = 2
_NS = 16
_NW = _NC * _NS
_LANES = 16
_EBLK = 80


def _pad_cols(a, cp):
    c = a.shape[-1]
    if c == cp:
        return a
    return jnp.pad(a, [(0, 0)] * (a.ndim - 1) + [(0, cp - c)])


def _rup128(c):
    return (c + 127) // 128 * 128


def _pick_bm(m):
    for bm in (512, 400, 256, 128, 64, 32, 16, 8):
        if m % bm == 0:
            return bm
    return m


def _mm_body(act, has_mul, *refs):
    if has_mul:
        a_ref, w_ref, b_ref, mul_ref, o_ref = refs
    else:
        a_ref, w_ref, b_ref, o_ref = refs
    acc = jnp.dot(a_ref[...], w_ref[...], preferred_element_type=jnp.float32)
    acc = acc + b_ref[...]
    if act == "elu":
        acc = jnp.where(acc > 0, acc, jnp.exp(jnp.minimum(acc, 0.0)) - 1.0)
    if has_mul:
        acc = acc * mul_ref[...]
    o_ref[...] = acc


def _mm(a, w, b=None, act=None, mul=None):
    m, k = a.shape
    n = w.shape[1]
    if b is None:
        b = jnp.zeros((n,), jnp.float32)
    bm = _pick_bm(m)
    in_specs = [
        pl.BlockSpec((bm, k), lambda i: (i, 0)),
        pl.BlockSpec((k, n), lambda i: (0, 0)),
        pl.BlockSpec((1, n), lambda i: (0, 0)),
    ]
    args = [a, w, b.reshape(1, n)]
    if mul is not None:
        in_specs.append(pl.BlockSpec((bm, 1), lambda i: (i, 0)))
        args.append(mul)
    return pl.pallas_call(
        functools.partial(_mm_body, act, mul is not None),
        grid=(m // bm,),
        in_specs=in_specs,
        out_specs=pl.BlockSpec((bm, n), lambda i: (i, 0)),
        out_shape=jax.ShapeDtypeStruct((m, n), jnp.float32),
    )(*args)


@functools.partial(jax.jit, static_argnames=("e", "cp"))
def _sc_gather(table, idx, e, cp):
    ew = e // _NW
    nb = ew // _EBLK
    nbuf = 2 if cp > 256 else 4
    rounds = (nb + nbuf - 1) // nbuf
    mesh = plsc.VectorSubcoreMesh(core_axis_name="c", subcore_axis_name="s")

    @functools.partial(
        pl.kernel,
        out_type=jax.ShapeDtypeStruct((e, cp), jnp.float32),
        mesh=mesh,
        scratch_types=(
            [pltpu.VMEM((_EBLK,), jnp.int32)] * nbuf
            + [pltpu.VMEM((_EBLK, cp), jnp.float32)] * nbuf
            + [pltpu.SemaphoreType.DMA] * (3 * nbuf)
        ),
    )
    def k(table_hbm, idx_hbm, out_hbm, *scr):
        idx_v = scr[0:nbuf]
        rows_v = scr[nbuf:2 * nbuf]
        isem = scr[2 * nbuf:3 * nbuf]
        gsem = scr[3 * nbuf:4 * nbuf]
        ssem = scr[4 * nbuf:5 * nbuf]
        cid = lax.axis_index("c")
        sid = lax.axis_index("s")
        base = (cid * _NS + sid) * ew

        def ebs(i):
            return pl.ds(pl.multiple_of(base + i * _EBLK, 8), _EBLK)

        for b in range(nbuf):
            pltpu.async_copy(idx_hbm.at[ebs(b)], idx_v[b], isem[b]).wait()
            pltpu.async_copy(table_hbm.at[idx_v[b]], rows_v[b], gsem[b])

        def step(t, carry):
            for b in range(nbuf):
                i = t * nbuf + b

                @pl.when(i < nb)
                def _():
                    pltpu.make_async_copy(
                        table_hbm.at[idx_v[b]], rows_v[b], gsem[b]).wait()
                    pltpu.async_copy(rows_v[b], out_hbm.at[ebs(i)], ssem[b])
                    j = i + nbuf

                    @pl.when(j < nb)
                    def _():
                        pltpu.async_copy(
                            idx_hbm.at[ebs(j)], idx_v[b], isem[b]).wait()
                        pltpu.make_async_copy(
                            rows_v[b], out_hbm.at[ebs(i)], ssem[b]).wait()
                        pltpu.async_copy(
                            table_hbm.at[idx_v[b]], rows_v[b], gsem[b])

            return carry

        lax.fori_loop(0, rounds, step, 0)
        for b in range(nbuf):
            pltpu.make_async_copy(rows_v[b], out_hbm.at[ebs(0)], ssem[b]).wait()

    return k(table, idx)


@functools.partial(jax.jit, static_argnames=("nseg", "cp"))
def _sc_segsum(vals, idx, nseg, cp):
    e = vals.shape[0]
    ew = e // _NW
    nb = ew // _EBLK
    ch = 40
    nch = nseg // ch
    rounds = (nch + _NS - 1) // _NS
    mesh = plsc.VectorSubcoreMesh(core_axis_name="c", subcore_axis_name="s")

    nbuf = 4
    @functools.partial(
        pl.kernel,
        out_type=jax.ShapeDtypeStruct((_NC, nseg, cp), jnp.float32),
        mesh=mesh,
        scratch_types=(
            [pltpu.VMEM_SHARED((nseg, cp), jnp.float32)]
            + [pltpu.VMEM((_EBLK,), jnp.int32)] * nbuf
            + [pltpu.VMEM((_EBLK, cp), jnp.float32)] * nbuf
            + [pltpu.VMEM((ch, cp), jnp.float32)]
            + [pltpu.SemaphoreType.DMA] * (3 * nbuf)
        ),
    )
    def k(vals_hbm, idx_hbm, out_hbm, acc_sh, *scr):
        idx_v = scr[0:nbuf]
        vals_v = scr[nbuf:2 * nbuf]
        bounce_v = scr[2 * nbuf]
        isem = scr[2 * nbuf + 1:3 * nbuf + 1]
        vsem = scr[3 * nbuf + 1:4 * nbuf + 1]
        asem = scr[4 * nbuf + 1:5 * nbuf + 1]
        cid = lax.axis_index("c")
        sid = lax.axis_index("s")
        base = (cid * _NS + sid) * ew

        def ebs(i):
            return pl.ds(pl.multiple_of(base + i * _EBLK, 8), _EBLK)

        for b in range(nbuf):
            pltpu.async_copy(idx_hbm.at[ebs(b)], idx_v[b], isem[b])
            pltpu.async_copy(vals_hbm.at[ebs(b)], vals_v[b], vsem[b])

        zero16 = jnp.zeros((_LANES,), jnp.float32)

        def zrow(r, carry):
            for c in range(cp // _LANES):
                bounce_v[r, pl.ds(c * _LANES, _LANES)] = zero16
            return carry

        lax.fori_loop(0, ch, zrow, 0)

        def zchunk(t, carry):
            j = t * _NS + sid

            @pl.when(j < nch)
            def _():
                r0 = pl.multiple_of(j * ch, 8)
                pltpu.sync_copy(bounce_v, acc_sh.at[pl.ds(r0, ch)])

            return carry

        lax.fori_loop(0, rounds, zchunk, 0)
        plsc.subcore_barrier()

        def step(t, carry):
            for b in range(nbuf):
                i = t * nbuf + b

                @pl.when(i < nb)
                def _():
                    pltpu.make_async_copy(
                        idx_hbm.at[ebs(i)], idx_v[b], isem[b]).wait()
                    pltpu.make_async_copy(
                        vals_hbm.at[ebs(i)], vals_v[b], vsem[b]).wait()
                    pltpu.async_copy(
                        vals_v[b], acc_sh.at[idx_v[b]], asem[b], add=True)
                    j = i + nbuf

                    @pl.when(j < nb)
                    def _():
                        pltpu.make_async_copy(
                            vals_v[b], acc_sh.at[idx_v[b]], asem[b]).wait()
                        pltpu.async_copy(idx_hbm.at[ebs(j)], idx_v[b], isem[b])
                        pltpu.async_copy(vals_hbm.at[ebs(j)], vals_v[b], vsem[b])

            return carry

        lax.fori_loop(0, (nb + nbuf - 1) // nbuf, step, 0)
        for b in range(nbuf):
            pltpu.make_async_copy(
                vals_v[b], acc_sh.at[idx_v[b]], asem[b]).wait()
        plsc.subcore_barrier()

        def ochunk(t, carry):
            j = t * _NS + sid

            @pl.when(j < nch)
            def _():
                r0 = pl.multiple_of(j * ch, 8)
                pltpu.sync_copy(acc_sh.at[pl.ds(r0, ch)], bounce_v)
                pltpu.sync_copy(bounce_v, out_hbm.at[cid, pl.ds(r0, ch)])

            return carry

        lax.fori_loop(0, rounds, ochunk, 0)

    parts = k(vals, idx)
    return parts[0] + parts[1]


@functools.partial(jax.jit, static_argnames=("nseg",))
def _sc_conv_pass(edge_h, xh, src, dst, nseg):
    e, cp = edge_h.shape
    ew = e // _NW
    nb = ew // _EBLK
    nbuf = 2
    ch = 40
    nch = nseg // ch
    rounds = (nch + _NS - 1) // _NS
    mesh = plsc.VectorSubcoreMesh(core_axis_name="c", subcore_axis_name="s")

    @functools.partial(
        pl.kernel,
        out_type=jax.ShapeDtypeStruct((_NC, nseg, cp), jnp.float32),
        mesh=mesh,
        scratch_types=(
            [pltpu.VMEM_SHARED((nseg, cp), jnp.float32)]
            + [pltpu.VMEM((_EBLK,), jnp.int32)] * (2 * nbuf)
            + [pltpu.VMEM((_EBLK, cp), jnp.float32)] * (2 * nbuf)
            + [pltpu.VMEM((ch, cp), jnp.float32)]
            + [pltpu.SemaphoreType.DMA] * (5 * nbuf)
        ),
    )
    def k(edge_hbm, x_hbm, src_hbm, dst_hbm, out_hbm, acc_sh, *scr):
        sidx = scr[0:nbuf]
        didx = scr[nbuf:2 * nbuf]
        ev = scr[2 * nbuf:3 * nbuf]
        rv = scr[3 * nbuf:4 * nbuf]
        bounce_v = scr[4 * nbuf]
        o = 4 * nbuf + 1
        ssem = scr[o:o + nbuf]
        dsem = scr[o + nbuf:o + 2 * nbuf]
        esem = scr[o + 2 * nbuf:o + 3 * nbuf]
        gsem = scr[o + 3 * nbuf:o + 4 * nbuf]
        asem = scr[o + 4 * nbuf:o + 5 * nbuf]
        cid = lax.axis_index("c")
        sid = lax.axis_index("s")
        base = (cid * _NS + sid) * ew

        def ebs(i):
            return pl.ds(pl.multiple_of(base + i * _EBLK, 8), _EBLK)

        for b in range(nbuf):
            pltpu.async_copy(src_hbm.at[ebs(b)], sidx[b], ssem[b])
            pltpu.async_copy(dst_hbm.at[ebs(b)], didx[b], dsem[b])
            pltpu.async_copy(edge_hbm.at[ebs(b)], ev[b], esem[b])

        zero16 = jnp.zeros((_LANES,), jnp.float32)

        def zrow(r, carry):
            for c in range(cp // _LANES):
                bounce_v[r, pl.ds(c * _LANES, _LANES)] = zero16
            return carry

        lax.fori_loop(0, ch, zrow, 0)

        def zchunk(t, carry):
            j = t * _NS + sid

            @pl.when(j < nch)
            def _():
                r0 = pl.multiple_of(j * ch, 8)
                pltpu.sync_copy(bounce_v, acc_sh.at[pl.ds(r0, ch)])

            return carry

        lax.fori_loop(0, rounds, zchunk, 0)
        plsc.subcore_barrier()

        for b in range(nbuf):
            pltpu.make_async_copy(dst_hbm.at[ebs(b)], didx[b], dsem[b]).wait()
            pltpu.async_copy(x_hbm.at[didx[b]], rv[b], gsem[b])

        def step(t, carry):
            for b in range(nbuf):
                i = t * nbuf + b

                @pl.when(i < nb)
                def _():
                    pltpu.make_async_copy(
                        edge_hbm.at[ebs(i)], ev[b], esem[b]).wait()
                    pltpu.make_async_copy(
                        x_hbm.at[didx[b]], rv[b], gsem[b]).wait()

                    def mulrow(r, carry2):
                        for c in range(cp // _LANES):
                            s = pl.ds(c * _LANES, _LANES)
                            ev[b][r, s] = ev[b][r, s] * rv[b][r, s]
                        return carry2

                    lax.fori_loop(0, _EBLK, mulrow, 0)
                    pltpu.make_async_copy(
                        src_hbm.at[ebs(i)], sidx[b], ssem[b]).wait()
                    pltpu.async_copy(
                        ev[b], acc_sh.at[sidx[b]], asem[b], add=True)
                    j = i + nbuf

                    @pl.when(j < nb)
                    def _():
                        pltpu.make_async_copy(
                            ev[b], acc_sh.at[sidx[b]], asem[b]).wait()
                        pltpu.async_copy(src_hbm.at[ebs(j)], sidx[b], ssem[b])
                        pltpu.async_copy(edge_hbm.at[ebs(j)], ev[b], esem[b])
                        pltpu.async_copy(
                            dst_hbm.at[ebs(j)], didx[b], dsem[b]).wait()
                        pltpu.async_copy(x_hbm.at[didx[b]], rv[b], gsem[b])

            return carry

        lax.fori_loop(0, (nb + nbuf - 1) // nbuf, step, 0)
        for b in range(nbuf):
            pltpu.make_async_copy(ev[b], acc_sh.at[sidx[b]], asem[b]).wait()
        plsc.subcore_barrier()

        def ochunk(t, carry):
            j = t * _NS + sid

            @pl.when(j < nch)
            def _():
                r0 = pl.multiple_of(j * ch, 8)
                pltpu.sync_copy(acc_sh.at[pl.ds(r0, ch)], bounce_v)
                pltpu.sync_copy(bounce_v, out_hbm.at[cid, pl.ds(r0, ch)])

            return carry

        lax.fori_loop(0, rounds, ochunk, 0)

    parts = k(edge_h, xh, src, dst)
    return parts[0] + parts[1]


def _segsum_wide(vals, idx, nseg):
    cp = vals.shape[1]
    parts = [_sc_segsum(vals[:, c:c + 128], idx, nseg, 128)
             for c in range(0, cp, 128)]
    return parts[0] if len(parts) == 1 else jnp.concatenate(parts, axis=1)


def kernel(x, edge_attr, params, edge_index, batch):
    n, _ = x.shape
    e = edge_index.shape[1]
    src, dst = edge_index[0], edge_index[1]
    mask = (edge_attr[:, 0:1] < 8).astype(jnp.float32)

    out = x
    n_layers = sum(1 for k_ in params if k_.startswith("conv"))
    for i in range(n_layers):
        p = params["conv%d" % i]
        cin = p["Wu"].shape[0]
        cp = _rup128(cin)
        we_p = _pad_cols(p["We"], cp)
        be_p = _pad_cols(p["be"], cp)
        xpad = _pad_cols(out, cp)
        cols = []
        for c0 in range(0, cp, 128):
            edge_h = _mm(edge_attr, we_p[:, c0:c0 + 128],
                         be_p[c0:c0 + 128], act="elu", mul=mask)
            cols.append(_sc_conv_pass(edge_h, xpad[:, c0:c0 + 128],
                                      src, dst, n))
        aggr = (cols[0] if len(cols) == 1
                else jnp.concatenate(cols, axis=1))[:, :cin]
        aggr = aggr + jax.nn.elu(p["be"])[None, :] * out
        out = _mm(out + aggr, p["Wu"], p["bu"])
        g, b = params["bn%d_g" % i], params["bn%d_b" % i]
        mu = out.mean(axis=0)
        var = out.var(axis=0)
        out = g * (out - mu) / jnp.sqrt(var + 1e-5) + b

    x0 = out
    c = x0.shape[1]
    cp = _rup128(c)
    nheads = sum(1 for k_ in params if k_.startswith("att"))

    wcols, bcols = [], []
    for j in range(nheads):
        p = params["att%d" % j]
        wf1, wf2, wf3 = p["Wf"][:c], p["Wf"][c:2 * c], p["Wf"][2 * c:3 * c]
        u = p["Wk"] * p["Wa"][:, 0][None, :]
        wv = jnp.dot(p["Wq"], u.T)
        wcols += [_pad_cols(wf1 + wf3, cp), _pad_cols(wv, cp),
                  _pad_cols(wf2 - wf3, cp)]
        bcols += [jnp.zeros((3 * cp,), jnp.float32)]
    wcat = jnp.concatenate(wcols, axis=1)
    nodetab = _mm(x0, wcat, jnp.concatenate(bcols))

    w4 = jnp.concatenate(
        [_pad_cols(params["att%d" % j]["Wf"][3 * c:], cp) for j in range(nheads)],
        axis=1)
    b4 = jnp.concatenate(
        [_pad_cols(params["att%d" % j]["bf"], cp) for j in range(nheads)])
    eaf = _mm(edge_attr, w4, b4)

    scale = c ** -0.5
    f_list, a_cols, fs_list, as_cols = [], [], [], []
    for j in range(nheads):
        av = nodetab[:, j * 3 * cp:(j * 3 + 2) * cp]
        bt = nodetab[:, (j * 3 + 2) * cp:(j + 1) * 3 * cp]
        g_av = _sc_gather(av, src, e, 2 * cp)
        g_b = _sc_gather(bt, dst, e, cp)
        g_a, g_v = g_av[:, :cp], g_av[:, cp:]
        pre = g_a + g_b + eaf[:, j * cp:(j + 1) * cp]
        f = jnp.where(pre > 0, pre, jnp.expm1(pre)) * mask
        a = jnp.tanh(scale * jnp.sum(f * g_v, axis=1))
        pre_s = nodetab[:, j * 3 * cp:j * 3 * cp + cp] + bt \
            + _pad_cols(params["att%d" % j]["bf"], cp)[None, :]
        f_self = jnp.where(pre_s > 0, pre_s, jnp.expm1(pre_s))
        vtab = nodetab[:, (j * 3 + 1) * cp:(j * 3 + 2) * cp]
        a_self = jnp.tanh(scale * jnp.sum(f_self * vtab, axis=1))
        f_list.append(f)
        fs_list.append(f_self)
        a_cols.append(a)
        as_cols.append(a_self)

    a128 = jnp.zeros((e, 128), jnp.float32)
    for j in range(nheads):
        a128 = a128.at[:, j].set(a_cols[j])
    suma = _sc_segsum(a128, src, n, 128)
    for j in range(nheads):
        suma = suma.at[:, j].add(as_cols[j])
    g_suma = _sc_gather(suma, src, e, 128)

    heads = []
    for j in range(nheads):
        p = params["att%d" % j]
        z = jnp.exp(a_cols[j] - g_suma[:, j])[:, None] * f_list[j]
        aggr = _segsum_wide(z, src, n)[:, :c]
        aggr = aggr + jnp.exp(as_cols[j] - suma[:, j])[:, None] \
            * fs_list[j][:, :c]
        o = _mm(x0 + aggr, p["Wu"], p["bu"])
        g, b = params["bn2_%d_g" % j], params["bn2_%d_b" % j]
        mu = o.mean(axis=0)
        var = o.var(axis=0)
        heads.append(g * (o - mu) / jnp.sqrt(var + 1e-5) + b)

    out = jnp.concatenate(heads, axis=1)
    ngraphs = 64
    sums = jax.ops.segment_sum(out, batch, num_segments=ngraphs)
    cnt = jax.ops.segment_sum(jnp.ones((n, 1), out.dtype), batch,
                              num_segments=ngraphs)
    pooled = sums / jnp.maximum(cnt, 1.0)
    h = _mm(pooled, params["W1"], params["b1"])
    h = jnp.where(h >= 0, h, params["prelu_a"] * h)
    h = jnp.dot(h, params["W2"]) + params["b2"]
    return h.reshape(-1)


# speedup vs baseline: 3.5713x; 1.0980x over previous
import functools

import jax
import jax.numpy as jnp
from jax import lax
from jax.experimental import pallas as pl
from jax.experimental.pallas import tpu as pltpu
from jax.experimental.pallas import tpu_sc as plsc

_NC = 2
_NS = 16
_NW = _NC * _NS
_LANES = 16
_EBLK = 80


def _pad_cols(a, cp):
    c = a.shape[-1]
    if c == cp:
        return a
    return jnp.pad(a, [(0, 0)] * (a.ndim - 1) + [(0, cp - c)])


def _rup128(c):
    return (c + 127) // 128 * 128


def _pick_bm(m):
    for bm in (512, 400, 256, 128, 64, 32, 16, 8):
        if m % bm == 0:
            return bm
    return m


def _mm_body(act, has_mul, *refs):
    if has_mul:
        a_ref, w_ref, b_ref, mul_ref, o_ref = refs
    else:
        a_ref, w_ref, b_ref, o_ref = refs
    acc = jnp.dot(a_ref[...], w_ref[...], preferred_element_type=jnp.float32)
    acc = acc + b_ref[...]
    if act == "elu":
        acc = jnp.where(acc > 0, acc, jnp.exp(jnp.minimum(acc, 0.0)) - 1.0)
    if has_mul:
        acc = acc * mul_ref[...]
    o_ref[...] = acc


def _mm(a, w, b=None, act=None, mul=None):
    m, k = a.shape
    n = w.shape[1]
    if b is None:
        b = jnp.zeros((n,), jnp.float32)
    bm = _pick_bm(m)
    in_specs = [
        pl.BlockSpec((bm, k), lambda i: (i, 0)),
        pl.BlockSpec((k, n), lambda i: (0, 0)),
        pl.BlockSpec((1, n), lambda i: (0, 0)),
    ]
    args = [a, w, b.reshape(1, n)]
    if mul is not None:
        in_specs.append(pl.BlockSpec((bm, 1), lambda i: (i, 0)))
        args.append(mul)
    return pl.pallas_call(
        functools.partial(_mm_body, act, mul is not None),
        grid=(m // bm,),
        in_specs=in_specs,
        out_specs=pl.BlockSpec((bm, n), lambda i: (i, 0)),
        out_shape=jax.ShapeDtypeStruct((m, n), jnp.float32),
    )(*args)


@functools.partial(jax.jit, static_argnames=("e", "cp"))
def _sc_gather(table, idx, e, cp):
    ew = e // _NW
    nb = ew // _EBLK
    nbuf = 2 if cp > 256 else 4
    rounds = (nb + nbuf - 1) // nbuf
    mesh = plsc.VectorSubcoreMesh(core_axis_name="c", subcore_axis_name="s")

    @functools.partial(
        pl.kernel,
        out_type=jax.ShapeDtypeStruct((e, cp), jnp.float32),
        mesh=mesh,
        scratch_types=(
            [pltpu.VMEM((_EBLK,), jnp.int32)] * nbuf
            + [pltpu.VMEM((_EBLK, cp), jnp.float32)] * nbuf
            + [pltpu.SemaphoreType.DMA] * (3 * nbuf)
        ),
    )
    def k(table_hbm, idx_hbm, out_hbm, *scr):
        idx_v = scr[0:nbuf]
        rows_v = scr[nbuf:2 * nbuf]
        isem = scr[2 * nbuf:3 * nbuf]
        gsem = scr[3 * nbuf:4 * nbuf]
        ssem = scr[4 * nbuf:5 * nbuf]
        cid = lax.axis_index("c")
        sid = lax.axis_index("s")
        base = (cid * _NS + sid) * ew

        def ebs(i):
            return pl.ds(pl.multiple_of(base + i * _EBLK, 8), _EBLK)

        for b in range(nbuf):
            pltpu.async_copy(idx_hbm.at[ebs(b)], idx_v[b], isem[b]).wait()
            pltpu.async_copy(table_hbm.at[idx_v[b]], rows_v[b], gsem[b])

        def step(t, carry):
            for b in range(nbuf):
                i = t * nbuf + b

                @pl.when(i < nb)
                def _():
                    pltpu.make_async_copy(
                        table_hbm.at[idx_v[b]], rows_v[b], gsem[b]).wait()
                    pltpu.async_copy(rows_v[b], out_hbm.at[ebs(i)], ssem[b])
                    j = i + nbuf

                    @pl.when(j < nb)
                    def _():
                        pltpu.async_copy(
                            idx_hbm.at[ebs(j)], idx_v[b], isem[b]).wait()
                        pltpu.make_async_copy(
                            rows_v[b], out_hbm.at[ebs(i)], ssem[b]).wait()
                        pltpu.async_copy(
                            table_hbm.at[idx_v[b]], rows_v[b], gsem[b])

            return carry

        lax.fori_loop(0, rounds, step, 0)
        for b in range(nbuf):
            pltpu.make_async_copy(rows_v[b], out_hbm.at[ebs(0)], ssem[b]).wait()

    return k(table, idx)


@functools.partial(jax.jit, static_argnames=("nseg", "cp"))
def _sc_segsum(vals, idx, nseg, cp):
    e = vals.shape[0]
    ew = e // _NW
    nb = ew // _EBLK
    ch = 40
    nch = nseg // ch
    rounds = (nch + _NS - 1) // _NS
    mesh = plsc.VectorSubcoreMesh(core_axis_name="c", subcore_axis_name="s")

    nbuf = 4
    @functools.partial(
        pl.kernel,
        out_type=jax.ShapeDtypeStruct((_NC, nseg, cp), jnp.float32),
        mesh=mesh,
        scratch_types=(
            [pltpu.VMEM_SHARED((nseg, cp), jnp.float32)]
            + [pltpu.VMEM((_EBLK,), jnp.int32)] * nbuf
            + [pltpu.VMEM((_EBLK, cp), jnp.float32)] * nbuf
            + [pltpu.VMEM((ch, cp), jnp.float32)]
            + [pltpu.SemaphoreType.DMA] * (3 * nbuf)
        ),
    )
    def k(vals_hbm, idx_hbm, out_hbm, acc_sh, *scr):
        idx_v = scr[0:nbuf]
        vals_v = scr[nbuf:2 * nbuf]
        bounce_v = scr[2 * nbuf]
        isem = scr[2 * nbuf + 1:3 * nbuf + 1]
        vsem = scr[3 * nbuf + 1:4 * nbuf + 1]
        asem = scr[4 * nbuf + 1:5 * nbuf + 1]
        cid = lax.axis_index("c")
        sid = lax.axis_index("s")
        base = (cid * _NS + sid) * ew

        def ebs(i):
            return pl.ds(pl.multiple_of(base + i * _EBLK, 8), _EBLK)

        for b in range(nbuf):
            pltpu.async_copy(idx_hbm.at[ebs(b)], idx_v[b], isem[b])
            pltpu.async_copy(vals_hbm.at[ebs(b)], vals_v[b], vsem[b])

        zero16 = jnp.zeros((_LANES,), jnp.float32)

        def zrow(r, carry):
            for c in range(cp // _LANES):
                bounce_v[r, pl.ds(c * _LANES, _LANES)] = zero16
            return carry

        lax.fori_loop(0, ch, zrow, 0)

        def zchunk(t, carry):
            j = t * _NS + sid

            @pl.when(j < nch)
            def _():
                r0 = pl.multiple_of(j * ch, 8)
                pltpu.sync_copy(bounce_v, acc_sh.at[pl.ds(r0, ch)])

            return carry

        lax.fori_loop(0, rounds, zchunk, 0)
        plsc.subcore_barrier()

        def step(t, carry):
            for b in range(nbuf):
                i = t * nbuf + b

                @pl.when(i < nb)
                def _():
                    pltpu.make_async_copy(
                        idx_hbm.at[ebs(i)], idx_v[b], isem[b]).wait()
                    pltpu.make_async_copy(
                        vals_hbm.at[ebs(i)], vals_v[b], vsem[b]).wait()
                    pltpu.async_copy(
                        vals_v[b], acc_sh.at[idx_v[b]], asem[b], add=True)
                    j = i + nbuf

                    @pl.when(j < nb)
                    def _():
                        pltpu.make_async_copy(
                            vals_v[b], acc_sh.at[idx_v[b]], asem[b]).wait()
                        pltpu.async_copy(idx_hbm.at[ebs(j)], idx_v[b], isem[b])
                        pltpu.async_copy(vals_hbm.at[ebs(j)], vals_v[b], vsem[b])

            return carry

        lax.fori_loop(0, (nb + nbuf - 1) // nbuf, step, 0)
        for b in range(nbuf):
            pltpu.make_async_copy(
                vals_v[b], acc_sh.at[idx_v[b]], asem[b]).wait()
        plsc.subcore_barrier()

        def ochunk(t, carry):
            j = t * _NS + sid

            @pl.when(j < nch)
            def _():
                r0 = pl.multiple_of(j * ch, 8)
                pltpu.sync_copy(acc_sh.at[pl.ds(r0, ch)], bounce_v)
                pltpu.sync_copy(bounce_v, out_hbm.at[cid, pl.ds(r0, ch)])

            return carry

        lax.fori_loop(0, rounds, ochunk, 0)

    parts = k(vals, idx)
    return parts[0] + parts[1]


@functools.partial(jax.jit, static_argnames=("nseg",))
def _sc_conv_pass(edge_h, xh, src, dst, nseg):
    e, cp = edge_h.shape
    ew = e // _NW
    nb = ew // _EBLK
    nbuf = 2
    ch = 40
    nch = nseg // ch
    rounds = (nch + _NS - 1) // _NS
    mesh = plsc.VectorSubcoreMesh(core_axis_name="c", subcore_axis_name="s")

    @functools.partial(
        pl.kernel,
        out_type=jax.ShapeDtypeStruct((_NC, nseg, cp), jnp.float32),
        mesh=mesh,
        scratch_types=(
            [pltpu.VMEM_SHARED((nseg, cp), jnp.float32)]
            + [pltpu.VMEM((_EBLK,), jnp.int32)] * (2 * nbuf)
            + [pltpu.VMEM((_EBLK, cp), jnp.float32)] * (2 * nbuf)
            + [pltpu.VMEM((ch, cp), jnp.float32)]
            + [pltpu.SemaphoreType.DMA] * (5 * nbuf)
        ),
    )
    def k(edge_hbm, x_hbm, src_hbm, dst_hbm, out_hbm, acc_sh, *scr):
        sidx = scr[0:nbuf]
        didx = scr[nbuf:2 * nbuf]
        ev = scr[2 * nbuf:3 * nbuf]
        rv = scr[3 * nbuf:4 * nbuf]
        bounce_v = scr[4 * nbuf]
        o = 4 * nbuf + 1
        ssem = scr[o:o + nbuf]
        dsem = scr[o + nbuf:o + 2 * nbuf]
        esem = scr[o + 2 * nbuf:o + 3 * nbuf]
        gsem = scr[o + 3 * nbuf:o + 4 * nbuf]
        asem = scr[o + 4 * nbuf:o + 5 * nbuf]
        cid = lax.axis_index("c")
        sid = lax.axis_index("s")
        base = (cid * _NS + sid) * ew

        def ebs(i):
            return pl.ds(pl.multiple_of(base + i * _EBLK, 8), _EBLK)

        for b in range(nbuf):
            pltpu.async_copy(src_hbm.at[ebs(b)], sidx[b], ssem[b])
            pltpu.async_copy(dst_hbm.at[ebs(b)], didx[b], dsem[b])
            pltpu.async_copy(edge_hbm.at[ebs(b)], ev[b], esem[b])

        zero16 = jnp.zeros((_LANES,), jnp.float32)

        def zrow(r, carry):
            for c in range(cp // _LANES):
                bounce_v[r, pl.ds(c * _LANES, _LANES)] = zero16
            return carry

        lax.fori_loop(0, ch, zrow, 0)

        def zchunk(t, carry):
            j = t * _NS + sid

            @pl.when(j < nch)
            def _():
                r0 = pl.multiple_of(j * ch, 8)
                pltpu.sync_copy(bounce_v, acc_sh.at[pl.ds(r0, ch)])

            return carry

        lax.fori_loop(0, rounds, zchunk, 0)
        plsc.subcore_barrier()

        for b in range(nbuf):
            pltpu.make_async_copy(dst_hbm.at[ebs(b)], didx[b], dsem[b]).wait()
            pltpu.async_copy(x_hbm.at[didx[b]], rv[b], gsem[b])

        def step(t, carry):
            for b in range(nbuf):
                i = t * nbuf + b

                @pl.when(i < nb)
                def _():
                    pltpu.make_async_copy(
                        edge_hbm.at[ebs(i)], ev[b], esem[b]).wait()
                    pltpu.make_async_copy(
                        x_hbm.at[didx[b]], rv[b], gsem[b]).wait()

                    def mulrow(r, carry2):
                        for c in range(cp // _LANES):
                            s = pl.ds(c * _LANES, _LANES)
                            ev[b][r, s] = ev[b][r, s] * rv[b][r, s]
                        return carry2

                    lax.fori_loop(0, _EBLK, mulrow, 0)
                    pltpu.make_async_copy(
                        src_hbm.at[ebs(i)], sidx[b], ssem[b]).wait()
                    pltpu.async_copy(
                        ev[b], acc_sh.at[sidx[b]], asem[b], add=True)
                    j = i + nbuf

                    @pl.when(j < nb)
                    def _():
                        pltpu.make_async_copy(
                            ev[b], acc_sh.at[sidx[b]], asem[b]).wait()
                        pltpu.async_copy(src_hbm.at[ebs(j)], sidx[b], ssem[b])
                        pltpu.async_copy(edge_hbm.at[ebs(j)], ev[b], esem[b])
                        pltpu.async_copy(
                            dst_hbm.at[ebs(j)], didx[b], dsem[b]).wait()
                        pltpu.async_copy(x_hbm.at[didx[b]], rv[b], gsem[b])

            return carry

        lax.fori_loop(0, (nb + nbuf - 1) // nbuf, step, 0)
        for b in range(nbuf):
            pltpu.make_async_copy(ev[b], acc_sh.at[sidx[b]], asem[b]).wait()
        plsc.subcore_barrier()

        def ochunk(t, carry):
            j = t * _NS + sid

            @pl.when(j < nch)
            def _():
                r0 = pl.multiple_of(j * ch, 8)
                pltpu.sync_copy(acc_sh.at[pl.ds(r0, ch)], bounce_v)
                pltpu.sync_copy(bounce_v, out_hbm.at[cid, pl.ds(r0, ch)])

            return carry

        lax.fori_loop(0, rounds, ochunk, 0)

    parts = k(edge_h, xh, src, dst)
    return parts[0] + parts[1]


@functools.partial(jax.jit, static_argnames=("nseg",))
def _sc_wsum(fh, w, src, nseg):
    e, cp = fh.shape
    ew = e // _NW
    nb = ew // _EBLK
    nbuf = 4
    ch = 40
    nch = nseg // ch
    rounds = (nch + _NS - 1) // _NS
    mesh = plsc.VectorSubcoreMesh(core_axis_name="c", subcore_axis_name="s")

    @functools.partial(
        pl.kernel,
        out_type=jax.ShapeDtypeStruct((_NC, nseg, cp), jnp.float32),
        mesh=mesh,
        scratch_types=(
            [pltpu.VMEM_SHARED((nseg, cp), jnp.float32)]
            + [pltpu.VMEM((_EBLK,), jnp.int32)] * nbuf
            + [pltpu.VMEM((_EBLK,), jnp.float32)] * nbuf
            + [pltpu.VMEM((_EBLK, cp), jnp.float32)] * nbuf
            + [pltpu.VMEM((ch, cp), jnp.float32)]
            + [pltpu.SemaphoreType.DMA] * (4 * nbuf)
        ),
    )
    def k(f_hbm, w_hbm, src_hbm, out_hbm, acc_sh, *scr):
        sidx = scr[0:nbuf]
        wv = scr[nbuf:2 * nbuf]
        fv = scr[2 * nbuf:3 * nbuf]
        bounce_v = scr[3 * nbuf]
        o = 3 * nbuf + 1
        ssem = scr[o:o + nbuf]
        wsem = scr[o + nbuf:o + 2 * nbuf]
        fsem = scr[o + 2 * nbuf:o + 3 * nbuf]
        asem = scr[o + 3 * nbuf:o + 4 * nbuf]
        cid = lax.axis_index("c")
        sid = lax.axis_index("s")
        base = (cid * _NS + sid) * ew

        def ebs(i):
            return pl.ds(pl.multiple_of(base + i * _EBLK, 8), _EBLK)

        for b in range(nbuf):
            pltpu.async_copy(src_hbm.at[ebs(b)], sidx[b], ssem[b])
            pltpu.async_copy(w_hbm.at[ebs(b)], wv[b], wsem[b])
            pltpu.async_copy(f_hbm.at[ebs(b)], fv[b], fsem[b])

        zero16 = jnp.zeros((_LANES,), jnp.float32)

        def zrow(r, carry):
            for c in range(cp // _LANES):
                bounce_v[r, pl.ds(c * _LANES, _LANES)] = zero16
            return carry

        lax.fori_loop(0, ch, zrow, 0)

        def zchunk(t, carry):
            j = t * _NS + sid

            @pl.when(j < nch)
            def _():
                r0 = pl.multiple_of(j * ch, 8)
                pltpu.sync_copy(bounce_v, acc_sh.at[pl.ds(r0, ch)])

            return carry

        lax.fori_loop(0, rounds, zchunk, 0)
        plsc.subcore_barrier()

        def step(t, carry):
            for b in range(nbuf):
                i = t * nbuf + b

                @pl.when(i < nb)
                def _():
                    pltpu.make_async_copy(
                        f_hbm.at[ebs(i)], fv[b], fsem[b]).wait()
                    pltpu.make_async_copy(
                        w_hbm.at[ebs(i)], wv[b], wsem[b]).wait()

                    def mulgrp(g, carry2):
                        wvec = wv[b][pl.ds(g * _LANES, _LANES)]
                        for kk in range(_LANES):
                            r = g * _LANES + kk
                            for c in range(cp // _LANES):
                                s = pl.ds(c * _LANES, _LANES)
                                fv[b][r, s] = fv[b][r, s] * wvec[kk]
                        return carry2

                    lax.fori_loop(0, _EBLK // _LANES, mulgrp, 0)
                    pltpu.make_async_copy(
                        src_hbm.at[ebs(i)], sidx[b], ssem[b]).wait()
                    pltpu.async_copy(
                        fv[b], acc_sh.at[sidx[b]], asem[b], add=True)
                    j = i + nbuf

                    @pl.when(j < nb)
                    def _():
                        pltpu.make_async_copy(
                            fv[b], acc_sh.at[sidx[b]], asem[b]).wait()
                        pltpu.async_copy(src_hbm.at[ebs(j)], sidx[b], ssem[b])
                        pltpu.async_copy(w_hbm.at[ebs(j)], wv[b], wsem[b])
                        pltpu.async_copy(f_hbm.at[ebs(j)], fv[b], fsem[b])

            return carry

        lax.fori_loop(0, (nb + nbuf - 1) // nbuf, step, 0)
        for b in range(nbuf):
            pltpu.make_async_copy(fv[b], acc_sh.at[sidx[b]], asem[b]).wait()
        plsc.subcore_barrier()

        def ochunk(t, carry):
            j = t * _NS + sid

            @pl.when(j < nch)
            def _():
                r0 = pl.multiple_of(j * ch, 8)
                pltpu.sync_copy(acc_sh.at[pl.ds(r0, ch)], bounce_v)
                pltpu.sync_copy(bounce_v, out_hbm.at[cid, pl.ds(r0, ch)])

            return carry

        lax.fori_loop(0, rounds, ochunk, 0)

    parts = k(fh, w, src)
    return parts[0] + parts[1]


def _segsum_wide(vals, idx, nseg):
    cp = vals.shape[1]
    parts = [_sc_segsum(vals[:, c:c + 128], idx, nseg, 128)
             for c in range(0, cp, 128)]
    return parts[0] if len(parts) == 1 else jnp.concatenate(parts, axis=1)


def kernel(x, edge_attr, params, edge_index, batch):
    n, _ = x.shape
    e = edge_index.shape[1]
    src, dst = edge_index[0], edge_index[1]
    mask = (edge_attr[:, 0:1] < 8).astype(jnp.float32)

    out = x
    n_layers = sum(1 for k_ in params if k_.startswith("conv"))
    for i in range(n_layers):
        p = params["conv%d" % i]
        cin = p["Wu"].shape[0]
        cp = _rup128(cin)
        we_p = _pad_cols(p["We"], cp)
        be_p = _pad_cols(p["be"], cp)
        xpad = _pad_cols(out, cp)
        cols = []
        for c0 in range(0, cp, 128):
            edge_h = _mm(edge_attr, we_p[:, c0:c0 + 128],
                         be_p[c0:c0 + 128], act="elu", mul=mask)
            cols.append(_sc_conv_pass(edge_h, xpad[:, c0:c0 + 128],
                                      src, dst, n))
        aggr = (cols[0] if len(cols) == 1
                else jnp.concatenate(cols, axis=1))[:, :cin]
        aggr = aggr + jax.nn.elu(p["be"])[None, :] * out
        out = _mm(out + aggr, p["Wu"], p["bu"])
        g, b = params["bn%d_g" % i], params["bn%d_b" % i]
        mu = out.mean(axis=0)
        var = out.var(axis=0)
        out = g * (out - mu) / jnp.sqrt(var + 1e-5) + b

    x0 = out
    c = x0.shape[1]
    cp = _rup128(c)
    nheads = sum(1 for k_ in params if k_.startswith("att"))

    wcols, bcols = [], []
    for j in range(nheads):
        p = params["att%d" % j]
        wf1, wf2, wf3 = p["Wf"][:c], p["Wf"][c:2 * c], p["Wf"][2 * c:3 * c]
        u = p["Wk"] * p["Wa"][:, 0][None, :]
        wv = jnp.dot(p["Wq"], u.T)
        wcols += [_pad_cols(wf1 + wf3, cp), _pad_cols(wv, cp),
                  _pad_cols(wf2 - wf3, cp)]
        bcols += [jnp.zeros((3 * cp,), jnp.float32)]
    wcat = jnp.concatenate(wcols, axis=1)
    nodetab = _mm(x0, wcat, jnp.concatenate(bcols))

    w4 = jnp.concatenate(
        [_pad_cols(params["att%d" % j]["Wf"][3 * c:], cp) for j in range(nheads)],
        axis=1)
    b4 = jnp.concatenate(
        [_pad_cols(params["att%d" % j]["bf"], cp) for j in range(nheads)])
    eaf = _mm(edge_attr, w4, b4)

    scale = c ** -0.5
    f_list, a_cols, fs_list, as_cols = [], [], [], []
    for j in range(nheads):
        av = nodetab[:, j * 3 * cp:(j * 3 + 2) * cp]
        bt = nodetab[:, (j * 3 + 2) * cp:(j + 1) * 3 * cp]
        g_av = _sc_gather(av, src, e, 2 * cp)
        g_b = _sc_gather(bt, dst, e, cp)
        halves = []
        a_pre = 0.0
        for c0 in range(0, cp, 128):
            pre_h = (g_av[:, c0:c0 + 128] + g_b[:, c0:c0 + 128]
                     + eaf[:, j * cp + c0:j * cp + c0 + 128])
            f_h = jnp.where(pre_h > 0, pre_h,
                            jnp.exp(jnp.minimum(pre_h, 0.0)) - 1.0) * mask
            a_pre = a_pre + jnp.sum(f_h * g_av[:, cp + c0:cp + c0 + 128],
                                    axis=1)
            halves.append(f_h)
        a = jnp.tanh(scale * a_pre)
        pre_s = nodetab[:, j * 3 * cp:j * 3 * cp + cp] + bt \
            + _pad_cols(params["att%d" % j]["bf"], cp)[None, :]
        f_self = jnp.where(pre_s > 0, pre_s, jnp.expm1(pre_s))
        vtab = nodetab[:, (j * 3 + 1) * cp:(j * 3 + 2) * cp]
        a_self = jnp.tanh(scale * jnp.sum(f_self * vtab, axis=1))
        f_list.append(halves)
        fs_list.append(f_self)
        a_cols.append(a)
        as_cols.append(a_self)

    a128 = jnp.zeros((e, 128), jnp.float32)
    for j in range(nheads):
        a128 = a128.at[:, j].set(a_cols[j])
    suma = _sc_segsum(a128, src, n, 128)
    for j in range(nheads):
        suma = suma.at[:, j].add(as_cols[j])
    g_suma = _sc_gather(suma, src, e, 128)

    heads = []
    for j in range(nheads):
        p = params["att%d" % j]
        wj = jnp.exp(a_cols[j] - g_suma[:, j])
        parts = [_sc_wsum(fh, wj, src, n) for fh in f_list[j]]
        aggr = jnp.concatenate(parts, axis=1)[:, :c]
        aggr = aggr + jnp.exp(as_cols[j] - suma[:, j])[:, None] \
            * fs_list[j][:, :c]
        o = _mm(x0 + aggr, p["Wu"], p["bu"])
        g, b = params["bn2_%d_g" % j], params["bn2_%d_b" % j]
        mu = o.mean(axis=0)
        var = o.var(axis=0)
        heads.append(g * (o - mu) / jnp.sqrt(var + 1e-5) + b)

    out = jnp.concatenate(heads, axis=1)
    ngraphs = 64
    sums = jax.ops.segment_sum(out, batch, num_segments=ngraphs)
    cnt = jax.ops.segment_sum(jnp.ones((n, 1), out.dtype), batch,
                              num_segments=ngraphs)
    pooled = sums / jnp.maximum(cnt, 1.0)
    h = _mm(pooled, params["W1"], params["b1"])
    h = jnp.where(h >= 0, h, params["prelu_a"] * h)
    h = jnp.dot(h, params["W2"]) + params["b2"]
    return h.reshape(-1)


# speedup vs baseline: 3.5725x; 1.0003x over previous
import functools

import jax
import jax.numpy as jnp
from jax import lax
from jax.experimental import pallas as pl
from jax.experimental.pallas import tpu as pltpu
from jax.experimental.pallas import tpu_sc as plsc

_NC = 2
_NS = 16
_NW = _NC * _NS
_LANES = 16
_EBLK = 80


def _pad_cols(a, cp):
    c = a.shape[-1]
    if c == cp:
        return a
    return jnp.pad(a, [(0, 0)] * (a.ndim - 1) + [(0, cp - c)])


def _rup128(c):
    return (c + 127) // 128 * 128


def _pick_bm(m):
    for bm in (512, 400, 256, 128, 64, 32, 16, 8):
        if m % bm == 0:
            return bm
    return m


def _mm_body(act, has_mul, *refs):
    if has_mul:
        a_ref, w_ref, b_ref, mul_ref, o_ref = refs
    else:
        a_ref, w_ref, b_ref, o_ref = refs
    acc = jnp.dot(a_ref[...], w_ref[...], preferred_element_type=jnp.float32)
    acc = acc + b_ref[...]
    if act == "elu":
        acc = jnp.where(acc > 0, acc, jnp.exp(jnp.minimum(acc, 0.0)) - 1.0)
    if has_mul:
        acc = acc * mul_ref[...]
    o_ref[...] = acc


def _mm(a, w, b=None, act=None, mul=None):
    m, k = a.shape
    n = w.shape[1]
    if b is None:
        b = jnp.zeros((n,), jnp.float32)
    bm = _pick_bm(m)
    in_specs = [
        pl.BlockSpec((bm, k), lambda i: (i, 0)),
        pl.BlockSpec((k, n), lambda i: (0, 0)),
        pl.BlockSpec((1, n), lambda i: (0, 0)),
    ]
    args = [a, w, b.reshape(1, n)]
    if mul is not None:
        in_specs.append(pl.BlockSpec((bm, 1), lambda i: (i, 0)))
        args.append(mul)
    return pl.pallas_call(
        functools.partial(_mm_body, act, mul is not None),
        grid=(m // bm,),
        in_specs=in_specs,
        out_specs=pl.BlockSpec((bm, n), lambda i: (i, 0)),
        out_shape=jax.ShapeDtypeStruct((m, n), jnp.float32),
    )(*args)


@functools.partial(jax.jit, static_argnames=("e", "cp"))
def _sc_gather(table, idx, e, cp):
    ew = e // _NW
    nb = ew // _EBLK
    nbuf = 3 if cp > 256 else 6
    rounds = (nb + nbuf - 1) // nbuf
    mesh = plsc.VectorSubcoreMesh(core_axis_name="c", subcore_axis_name="s")

    @functools.partial(
        pl.kernel,
        out_type=jax.ShapeDtypeStruct((e, cp), jnp.float32),
        mesh=mesh,
        scratch_types=(
            [pltpu.VMEM((_EBLK,), jnp.int32)] * nbuf
            + [pltpu.VMEM((_EBLK, cp), jnp.float32)] * nbuf
            + [pltpu.SemaphoreType.DMA] * (3 * nbuf)
        ),
    )
    def k(table_hbm, idx_hbm, out_hbm, *scr):
        idx_v = scr[0:nbuf]
        rows_v = scr[nbuf:2 * nbuf]
        isem = scr[2 * nbuf:3 * nbuf]
        gsem = scr[3 * nbuf:4 * nbuf]
        ssem = scr[4 * nbuf:5 * nbuf]
        cid = lax.axis_index("c")
        sid = lax.axis_index("s")
        base = (cid * _NS + sid) * ew

        def ebs(i):
            return pl.ds(pl.multiple_of(base + i * _EBLK, 8), _EBLK)

        for b in range(nbuf):
            pltpu.async_copy(idx_hbm.at[ebs(b)], idx_v[b], isem[b]).wait()
            pltpu.async_copy(table_hbm.at[idx_v[b]], rows_v[b], gsem[b])

        def step(t, carry):
            for b in range(nbuf):
                i = t * nbuf + b

                @pl.when(i < nb)
                def _():
                    pltpu.make_async_copy(
                        table_hbm.at[idx_v[b]], rows_v[b], gsem[b]).wait()
                    pltpu.async_copy(rows_v[b], out_hbm.at[ebs(i)], ssem[b])
                    j = i + nbuf

                    @pl.when(j < nb)
                    def _():
                        pltpu.async_copy(
                            idx_hbm.at[ebs(j)], idx_v[b], isem[b]).wait()
                        pltpu.make_async_copy(
                            rows_v[b], out_hbm.at[ebs(i)], ssem[b]).wait()
                        pltpu.async_copy(
                            table_hbm.at[idx_v[b]], rows_v[b], gsem[b])

            return carry

        lax.fori_loop(0, rounds, step, 0)
        for b in range(nbuf):
            pltpu.make_async_copy(rows_v[b], out_hbm.at[ebs(0)], ssem[b]).wait()

    return k(table, idx)


@functools.partial(jax.jit, static_argnames=("nseg", "cp"))
def _sc_segsum(vals, idx, nseg, cp):
    e = vals.shape[0]
    ew = e // _NW
    nb = ew // _EBLK
    ch = 40
    nch = nseg // ch
    rounds = (nch + _NS - 1) // _NS
    mesh = plsc.VectorSubcoreMesh(core_axis_name="c", subcore_axis_name="s")

    nbuf = 4
    @functools.partial(
        pl.kernel,
        out_type=jax.ShapeDtypeStruct((_NC, nseg, cp), jnp.float32),
        mesh=mesh,
        scratch_types=(
            [pltpu.VMEM_SHARED((nseg, cp), jnp.float32)]
            + [pltpu.VMEM((_EBLK,), jnp.int32)] * nbuf
            + [pltpu.VMEM((_EBLK, cp), jnp.float32)] * nbuf
            + [pltpu.VMEM((ch, cp), jnp.float32)]
            + [pltpu.SemaphoreType.DMA] * (3 * nbuf)
        ),
    )
    def k(vals_hbm, idx_hbm, out_hbm, acc_sh, *scr):
        idx_v = scr[0:nbuf]
        vals_v = scr[nbuf:2 * nbuf]
        bounce_v = scr[2 * nbuf]
        isem = scr[2 * nbuf + 1:3 * nbuf + 1]
        vsem = scr[3 * nbuf + 1:4 * nbuf + 1]
        asem = scr[4 * nbuf + 1:5 * nbuf + 1]
        cid = lax.axis_index("c")
        sid = lax.axis_index("s")
        base = (cid * _NS + sid) * ew

        def ebs(i):
            return pl.ds(pl.multiple_of(base + i * _EBLK, 8), _EBLK)

        for b in range(nbuf):
            pltpu.async_copy(idx_hbm.at[ebs(b)], idx_v[b], isem[b])
            pltpu.async_copy(vals_hbm.at[ebs(b)], vals_v[b], vsem[b])

        zero16 = jnp.zeros((_LANES,), jnp.float32)

        def zrow(r, carry):
            for c in range(cp // _LANES):
                bounce_v[r, pl.ds(c * _LANES, _LANES)] = zero16
            return carry

        lax.fori_loop(0, ch, zrow, 0)

        def zchunk(t, carry):
            j = t * _NS + sid

            @pl.when(j < nch)
            def _():
                r0 = pl.multiple_of(j * ch, 8)
                pltpu.sync_copy(bounce_v, acc_sh.at[pl.ds(r0, ch)])

            return carry

        lax.fori_loop(0, rounds, zchunk, 0)
        plsc.subcore_barrier()

        def step(t, carry):
            for b in range(nbuf):
                i = t * nbuf + b

                @pl.when(i < nb)
                def _():
                    pltpu.make_async_copy(
                        idx_hbm.at[ebs(i)], idx_v[b], isem[b]).wait()
                    pltpu.make_async_copy(
                        vals_hbm.at[ebs(i)], vals_v[b], vsem[b]).wait()
                    pltpu.async_copy(
                        vals_v[b], acc_sh.at[idx_v[b]], asem[b], add=True)
                    j = i + nbuf

                    @pl.when(j < nb)
                    def _():
                        pltpu.make_async_copy(
                            vals_v[b], acc_sh.at[idx_v[b]], asem[b]).wait()
                        pltpu.async_copy(idx_hbm.at[ebs(j)], idx_v[b], isem[b])
                        pltpu.async_copy(vals_hbm.at[ebs(j)], vals_v[b], vsem[b])

            return carry

        lax.fori_loop(0, (nb + nbuf - 1) // nbuf, step, 0)
        for b in range(nbuf):
            pltpu.make_async_copy(
                vals_v[b], acc_sh.at[idx_v[b]], asem[b]).wait()
        plsc.subcore_barrier()

        def ochunk(t, carry):
            j = t * _NS + sid

            @pl.when(j < nch)
            def _():
                r0 = pl.multiple_of(j * ch, 8)
                pltpu.sync_copy(acc_sh.at[pl.ds(r0, ch)], bounce_v)
                pltpu.sync_copy(bounce_v, out_hbm.at[cid, pl.ds(r0, ch)])

            return carry

        lax.fori_loop(0, rounds, ochunk, 0)

    parts = k(vals, idx)
    return parts[0] + parts[1]


@functools.partial(jax.jit, static_argnames=("nseg",))
def _sc_conv_pass(edge_h, xh, src, dst, nseg):
    e, cp = edge_h.shape
    ew = e // _NW
    nb = ew // _EBLK
    nbuf = 2
    ch = 40
    nch = nseg // ch
    rounds = (nch + _NS - 1) // _NS
    mesh = plsc.VectorSubcoreMesh(core_axis_name="c", subcore_axis_name="s")

    @functools.partial(
        pl.kernel,
        out_type=jax.ShapeDtypeStruct((_NC, nseg, cp), jnp.float32),
        mesh=mesh,
        scratch_types=(
            [pltpu.VMEM_SHARED((nseg, cp), jnp.float32)]
            + [pltpu.VMEM((_EBLK,), jnp.int32)] * (2 * nbuf)
            + [pltpu.VMEM((_EBLK, cp), jnp.float32)] * (2 * nbuf)
            + [pltpu.VMEM((ch, cp), jnp.float32)]
            + [pltpu.SemaphoreType.DMA] * (5 * nbuf)
        ),
    )
    def k(edge_hbm, x_hbm, src_hbm, dst_hbm, out_hbm, acc_sh, *scr):
        sidx = scr[0:nbuf]
        didx = scr[nbuf:2 * nbuf]
        ev = scr[2 * nbuf:3 * nbuf]
        rv = scr[3 * nbuf:4 * nbuf]
        bounce_v = scr[4 * nbuf]
        o = 4 * nbuf + 1
        ssem = scr[o:o + nbuf]
        dsem = scr[o + nbuf:o + 2 * nbuf]
        esem = scr[o + 2 * nbuf:o + 3 * nbuf]
        gsem = scr[o + 3 * nbuf:o + 4 * nbuf]
        asem = scr[o + 4 * nbuf:o + 5 * nbuf]
        cid = lax.axis_index("c")
        sid = lax.axis_index("s")
        base = (cid * _NS + sid) * ew

        def ebs(i):
            return pl.ds(pl.multiple_of(base + i * _EBLK, 8), _EBLK)

        for b in range(nbuf):
            pltpu.async_copy(src_hbm.at[ebs(b)], sidx[b], ssem[b])
            pltpu.async_copy(dst_hbm.at[ebs(b)], didx[b], dsem[b])
            pltpu.async_copy(edge_hbm.at[ebs(b)], ev[b], esem[b])

        zero16 = jnp.zeros((_LANES,), jnp.float32)

        def zrow(r, carry):
            for c in range(cp // _LANES):
                bounce_v[r, pl.ds(c * _LANES, _LANES)] = zero16
            return carry

        lax.fori_loop(0, ch, zrow, 0)

        def zchunk(t, carry):
            j = t * _NS + sid

            @pl.when(j < nch)
            def _():
                r0 = pl.multiple_of(j * ch, 8)
                pltpu.sync_copy(bounce_v, acc_sh.at[pl.ds(r0, ch)])

            return carry

        lax.fori_loop(0, rounds, zchunk, 0)
        plsc.subcore_barrier()

        for b in range(nbuf):
            pltpu.make_async_copy(dst_hbm.at[ebs(b)], didx[b], dsem[b]).wait()
            pltpu.async_copy(x_hbm.at[didx[b]], rv[b], gsem[b])

        def step(t, carry):
            for b in range(nbuf):
                i = t * nbuf + b

                @pl.when(i < nb)
                def _():
                    pltpu.make_async_copy(
                        edge_hbm.at[ebs(i)], ev[b], esem[b]).wait()
                    pltpu.make_async_copy(
                        x_hbm.at[didx[b]], rv[b], gsem[b]).wait()

                    def mulrow(r, carry2):
                        for c in range(cp // _LANES):
                            s = pl.ds(c * _LANES, _LANES)
                            ev[b][r, s] = ev[b][r, s] * rv[b][r, s]
                        return carry2

                    lax.fori_loop(0, _EBLK, mulrow, 0)
                    pltpu.make_async_copy(
                        src_hbm.at[ebs(i)], sidx[b], ssem[b]).wait()
                    pltpu.async_copy(
                        ev[b], acc_sh.at[sidx[b]], asem[b], add=True)
                    j = i + nbuf

                    @pl.when(j < nb)
                    def _():
                        pltpu.make_async_copy(
                            ev[b], acc_sh.at[sidx[b]], asem[b]).wait()
                        pltpu.async_copy(src_hbm.at[ebs(j)], sidx[b], ssem[b])
                        pltpu.async_copy(edge_hbm.at[ebs(j)], ev[b], esem[b])
                        pltpu.async_copy(
                            dst_hbm.at[ebs(j)], didx[b], dsem[b]).wait()
                        pltpu.async_copy(x_hbm.at[didx[b]], rv[b], gsem[b])

            return carry

        lax.fori_loop(0, (nb + nbuf - 1) // nbuf, step, 0)
        for b in range(nbuf):
            pltpu.make_async_copy(ev[b], acc_sh.at[sidx[b]], asem[b]).wait()
        plsc.subcore_barrier()

        def ochunk(t, carry):
            j = t * _NS + sid

            @pl.when(j < nch)
            def _():
                r0 = pl.multiple_of(j * ch, 8)
                pltpu.sync_copy(acc_sh.at[pl.ds(r0, ch)], bounce_v)
                pltpu.sync_copy(bounce_v, out_hbm.at[cid, pl.ds(r0, ch)])

            return carry

        lax.fori_loop(0, rounds, ochunk, 0)

    parts = k(edge_h, xh, src, dst)
    return parts[0] + parts[1]


@functools.partial(jax.jit, static_argnames=("nseg",))
def _sc_wsum(fh, w, src, nseg):
    e, cp = fh.shape
    ew = e // _NW
    nb = ew // _EBLK
    nbuf = 4
    ch = 40
    nch = nseg // ch
    rounds = (nch + _NS - 1) // _NS
    mesh = plsc.VectorSubcoreMesh(core_axis_name="c", subcore_axis_name="s")

    @functools.partial(
        pl.kernel,
        out_type=jax.ShapeDtypeStruct((_NC, nseg, cp), jnp.float32),
        mesh=mesh,
        scratch_types=(
            [pltpu.VMEM_SHARED((nseg, cp), jnp.float32)]
            + [pltpu.VMEM((_EBLK,), jnp.int32)] * nbuf
            + [pltpu.VMEM((_EBLK,), jnp.float32)] * nbuf
            + [pltpu.VMEM((_EBLK, cp), jnp.float32)] * nbuf
            + [pltpu.VMEM((ch, cp), jnp.float32)]
            + [pltpu.SemaphoreType.DMA] * (4 * nbuf)
        ),
    )
    def k(f_hbm, w_hbm, src_hbm, out_hbm, acc_sh, *scr):
        sidx = scr[0:nbuf]
        wv = scr[nbuf:2 * nbuf]
        fv = scr[2 * nbuf:3 * nbuf]
        bounce_v = scr[3 * nbuf]
        o = 3 * nbuf + 1
        ssem = scr[o:o + nbuf]
        wsem = scr[o + nbuf:o + 2 * nbuf]
        fsem = scr[o + 2 * nbuf:o + 3 * nbuf]
        asem = scr[o + 3 * nbuf:o + 4 * nbuf]
        cid = lax.axis_index("c")
        sid = lax.axis_index("s")
        base = (cid * _NS + sid) * ew

        def ebs(i):
            return pl.ds(pl.multiple_of(base + i * _EBLK, 8), _EBLK)

        for b in range(nbuf):
            pltpu.async_copy(src_hbm.at[ebs(b)], sidx[b], ssem[b])
            pltpu.async_copy(w_hbm.at[ebs(b)], wv[b], wsem[b])
            pltpu.async_copy(f_hbm.at[ebs(b)], fv[b], fsem[b])

        zero16 = jnp.zeros((_LANES,), jnp.float32)

        def zrow(r, carry):
            for c in range(cp // _LANES):
                bounce_v[r, pl.ds(c * _LANES, _LANES)] = zero16
            return carry

        lax.fori_loop(0, ch, zrow, 0)

        def zchunk(t, carry):
            j = t * _NS + sid

            @pl.when(j < nch)
            def _():
                r0 = pl.multiple_of(j * ch, 8)
                pltpu.sync_copy(bounce_v, acc_sh.at[pl.ds(r0, ch)])

            return carry

        lax.fori_loop(0, rounds, zchunk, 0)
        plsc.subcore_barrier()

        def step(t, carry):
            for b in range(nbuf):
                i = t * nbuf + b

                @pl.when(i < nb)
                def _():
                    pltpu.make_async_copy(
                        f_hbm.at[ebs(i)], fv[b], fsem[b]).wait()
                    pltpu.make_async_copy(
                        w_hbm.at[ebs(i)], wv[b], wsem[b]).wait()

                    def mulgrp(g, carry2):
                        wvec = wv[b][pl.ds(g * _LANES, _LANES)]
                        for kk in range(_LANES):
                            r = g * _LANES + kk
                            for c in range(cp // _LANES):
                                s = pl.ds(c * _LANES, _LANES)
                                fv[b][r, s] = fv[b][r, s] * wvec[kk]
                        return carry2

                    lax.fori_loop(0, _EBLK // _LANES, mulgrp, 0)
                    pltpu.make_async_copy(
                        src_hbm.at[ebs(i)], sidx[b], ssem[b]).wait()
                    pltpu.async_copy(
                        fv[b], acc_sh.at[sidx[b]], asem[b], add=True)
                    j = i + nbuf

                    @pl.when(j < nb)
                    def _():
                        pltpu.make_async_copy(
                            fv[b], acc_sh.at[sidx[b]], asem[b]).wait()
                        pltpu.async_copy(src_hbm.at[ebs(j)], sidx[b], ssem[b])
                        pltpu.async_copy(w_hbm.at[ebs(j)], wv[b], wsem[b])
                        pltpu.async_copy(f_hbm.at[ebs(j)], fv[b], fsem[b])

            return carry

        lax.fori_loop(0, (nb + nbuf - 1) // nbuf, step, 0)
        for b in range(nbuf):
            pltpu.make_async_copy(fv[b], acc_sh.at[sidx[b]], asem[b]).wait()
        plsc.subcore_barrier()

        def ochunk(t, carry):
            j = t * _NS + sid

            @pl.when(j < nch)
            def _():
                r0 = pl.multiple_of(j * ch, 8)
                pltpu.sync_copy(acc_sh.at[pl.ds(r0, ch)], bounce_v)
                pltpu.sync_copy(bounce_v, out_hbm.at[cid, pl.ds(r0, ch)])

            return carry

        lax.fori_loop(0, rounds, ochunk, 0)

    parts = k(fh, w, src)
    return parts[0] + parts[1]


def _segsum_wide(vals, idx, nseg):
    cp = vals.shape[1]
    parts = [_sc_segsum(vals[:, c:c + 128], idx, nseg, 128)
             for c in range(0, cp, 128)]
    return parts[0] if len(parts) == 1 else jnp.concatenate(parts, axis=1)


def kernel(x, edge_attr, params, edge_index, batch):
    n, _ = x.shape
    e = edge_index.shape[1]
    src, dst = edge_index[0], edge_index[1]
    mask = (edge_attr[:, 0:1] < 8).astype(jnp.float32)

    out = x
    n_layers = sum(1 for k_ in params if k_.startswith("conv"))
    for i in range(n_layers):
        p = params["conv%d" % i]
        cin = p["Wu"].shape[0]
        cp = _rup128(cin)
        we_p = _pad_cols(p["We"], cp)
        be_p = _pad_cols(p["be"], cp)
        xpad = _pad_cols(out, cp)
        cols = []
        for c0 in range(0, cp, 128):
            edge_h = _mm(edge_attr, we_p[:, c0:c0 + 128],
                         be_p[c0:c0 + 128], act="elu", mul=mask)
            cols.append(_sc_conv_pass(edge_h, xpad[:, c0:c0 + 128],
                                      src, dst, n))
        aggr = (cols[0] if len(cols) == 1
                else jnp.concatenate(cols, axis=1))[:, :cin]
        aggr = aggr + jax.nn.elu(p["be"])[None, :] * out
        out = _mm(out + aggr, p["Wu"], p["bu"])
        g, b = params["bn%d_g" % i], params["bn%d_b" % i]
        mu = out.mean(axis=0)
        var = out.var(axis=0)
        out = g * (out - mu) / jnp.sqrt(var + 1e-5) + b

    x0 = out
    c = x0.shape[1]
    cp = _rup128(c)
    nheads = sum(1 for k_ in params if k_.startswith("att"))

    wcols, bcols = [], []
    for j in range(nheads):
        p = params["att%d" % j]
        wf1, wf2, wf3 = p["Wf"][:c], p["Wf"][c:2 * c], p["Wf"][2 * c:3 * c]
        u = p["Wk"] * p["Wa"][:, 0][None, :]
        wv = jnp.dot(p["Wq"], u.T)
        wcols += [_pad_cols(wf1 + wf3, cp), _pad_cols(wv, cp),
                  _pad_cols(wf2 - wf3, cp)]
        bcols += [jnp.zeros((3 * cp,), jnp.float32)]
    wcat = jnp.concatenate(wcols, axis=1)
    nodetab = _mm(x0, wcat, jnp.concatenate(bcols))

    w4 = jnp.concatenate(
        [_pad_cols(params["att%d" % j]["Wf"][3 * c:], cp) for j in range(nheads)],
        axis=1)
    b4 = jnp.concatenate(
        [_pad_cols(params["att%d" % j]["bf"], cp) for j in range(nheads)])
    eaf = _mm(edge_attr, w4, b4)

    scale = c ** -0.5
    f_list, a_cols, fs_list, as_cols = [], [], [], []
    for j in range(nheads):
        av = nodetab[:, j * 3 * cp:(j * 3 + 2) * cp]
        bt = nodetab[:, (j * 3 + 2) * cp:(j + 1) * 3 * cp]
        g_av = _sc_gather(av, src, e, 2 * cp)
        g_b = _sc_gather(bt, dst, e, cp)
        halves = []
        a_pre = 0.0
        for c0 in range(0, cp, 128):
            pre_h = (g_av[:, c0:c0 + 128] + g_b[:, c0:c0 + 128]
                     + eaf[:, j * cp + c0:j * cp + c0 + 128])
            f_h = jnp.where(pre_h > 0, pre_h,
                            jnp.exp(jnp.minimum(pre_h, 0.0)) - 1.0) * mask
            a_pre = a_pre + jnp.sum(f_h * g_av[:, cp + c0:cp + c0 + 128],
                                    axis=1)
            halves.append(f_h)
        a = jnp.tanh(scale * a_pre)
        pre_s = nodetab[:, j * 3 * cp:j * 3 * cp + cp] + bt \
            + _pad_cols(params["att%d" % j]["bf"], cp)[None, :]
        f_self = jnp.where(pre_s > 0, pre_s, jnp.expm1(pre_s))
        vtab = nodetab[:, (j * 3 + 1) * cp:(j * 3 + 2) * cp]
        a_self = jnp.tanh(scale * jnp.sum(f_self * vtab, axis=1))
        f_list.append(halves)
        fs_list.append(f_self)
        a_cols.append(a)
        as_cols.append(a_self)

    a128 = jnp.zeros((e, 128), jnp.float32)
    for j in range(nheads):
        a128 = a128.at[:, j].set(a_cols[j])
    suma = _sc_segsum(a128, src, n, 128)
    for j in range(nheads):
        suma = suma.at[:, j].add(as_cols[j])
    g_suma = _sc_gather(suma, src, e, 128)

    heads = []
    for j in range(nheads):
        p = params["att%d" % j]
        wj = jnp.exp(a_cols[j] - g_suma[:, j])
        parts = [_sc_wsum(fh, wj, src, n) for fh in f_list[j]]
        aggr = jnp.concatenate(parts, axis=1)[:, :c]
        aggr = aggr + jnp.exp(as_cols[j] - suma[:, j])[:, None] \
            * fs_list[j][:, :c]
        o = _mm(x0 + aggr, p["Wu"], p["bu"])
        g, b = params["bn2_%d_g" % j], params["bn2_%d_b" % j]
        mu = o.mean(axis=0)
        var = o.var(axis=0)
        heads.append(g * (o - mu) / jnp.sqrt(var + 1e-5) + b)

    out = jnp.concatenate(heads, axis=1)
    ngraphs = 64
    sums = jax.ops.segment_sum(out, batch, num_segments=ngraphs)
    cnt = jax.ops.segment_sum(jnp.ones((n, 1), out.dtype), batch,
                              num_segments=ngraphs)
    pooled = sums / jnp.maximum(cnt, 1.0)
    h = _mm(pooled, params["W1"], params["b1"])
    h = jnp.where(h >= 0, h, params["prelu_a"] * h)
    h = jnp.dot(h, params["W2"]) + params["b2"]
    return h.reshape(-1)
